# baseline (speedup 1.0000x reference)
import jax
import jax.numpy as jnp
from jax.experimental import pallas as pl
from jax.experimental.pallas import tpu as pltpu


import functools

_BK = 2048


def _gcn_block_kernel(nkb, ktail, a_ref, b_ref, o_ref, b_bf):
    m = pl.program_id(0)
    kb = pl.program_id(1)
    last = nkb - 1

    @pl.when(m == 0)
    def _():
        b_bf[pl.ds(kb * _BK, _BK), :] = b_ref[...].astype(jnp.bfloat16)

        @pl.when(kb == last)
        def _():
            b_bf[pl.ds(last * _BK + ktail, _BK - ktail), :] = jnp.zeros(
                (_BK - ktail, b_bf.shape[1]), jnp.bfloat16)

    a = a_ref[...].astype(jnp.bfloat16)

    lane = jax.lax.broadcasted_iota(jnp.int32, a.shape, 1)
    a = jnp.where((kb != last) | (lane < ktail), a, jnp.bfloat16(0))

    part = jnp.dot(a, b_bf[pl.ds(kb * _BK, _BK), :],
                   preferred_element_type=jnp.float32)

    @pl.when(kb == 0)
    def _():
        o_ref[...] = part

    @pl.when((kb > 0) & (kb < last))
    def _():
        o_ref[...] += part

    @pl.when((kb == last) & (last > 0))
    def _():
        acc = o_ref[...] + part
        o_ref[...] = jnp.where(acc >= 0, acc, 0.5 * acc)


def kernel(adj, embeds):
    n, kdim = adj.shape
    d = embeds.shape[1]
    bm = 400
    nkb = pl.cdiv(kdim, _BK)
    ktail = kdim - (nkb - 1) * _BK
    kpad = nkb * _BK
    return pl.pallas_call(
        functools.partial(_gcn_block_kernel, nkb, ktail),
        grid=(n // bm, nkb),
        in_specs=[
            pl.BlockSpec((bm, _BK), lambda m, kb: (m, kb)),
            pl.BlockSpec((_BK, d),
                         lambda m, kb: (jnp.where(m == 0, kb, nkb - 1), 0)),
        ],
        out_specs=pl.BlockSpec((bm, d), lambda m, kb: (m, 0)),
        out_shape=jax.ShapeDtypeStruct((n, d), jnp.float32),
        scratch_shapes=[pltpu.VMEM((kpad, d), jnp.bfloat16)],
    )(adj, embeds)


# speedup vs baseline: 1.4055x; 1.4055x over previous
import jax
import jax.numpy as jnp
from jax.experimental import pallas as pl
from jax.experimental.pallas import tpu as pltpu


def _gcn_block_kernel(a_ref, b_ref, o_ref, b_bf):
    @pl.when(pl.program_id(0) == 0)
    def _():
        b_bf[...] = b_ref[...].astype(jnp.bfloat16)

    a = a_ref[...].astype(jnp.bfloat16)
    acc = jnp.dot(a, b_bf[...], preferred_element_type=jnp.float32)
    o_ref[...] = jnp.where(acc >= 0, acc, 0.5 * acc)


def kernel(adj, embeds):
    n, kdim = adj.shape
    d = embeds.shape[1]
    bm = 360
    return pl.pallas_call(
        _gcn_block_kernel,
        grid=(pl.cdiv(n, bm),),
        in_specs=[
            pl.BlockSpec((bm, kdim), lambda m: (m, 0)),
            pl.BlockSpec((kdim, d), lambda m: (0, 0)),
        ],
        out_specs=pl.BlockSpec((bm, d), lambda m: (m, 0)),
        out_shape=jax.ShapeDtypeStruct((n, d), jnp.float32),
        scratch_shapes=[pltpu.VMEM((kdim, d), jnp.bfloat16)],
    )(adj, embeds)
